# Initial kernel scaffold; baseline (speedup 1.0000x reference)
import jax
import jax.numpy as jnp
from jax.experimental import pallas as pl


def kernel(frequency):
    raise NotImplementedError("write your pallas kernel here")


# speedup vs baseline: 1.4041x; 1.4041x over previous
import functools

import jax
import jax.numpy as jnp
from jax import lax
from jax.experimental import pallas as pl
from jax.experimental.pallas import tpu as pltpu
from jax.experimental.pallas import tpu_sc as plsc

N = 16777216
NC, NS, L = 2, 16, 16
NW = NC * NS
PW = N // NW
C = 16384
NCHUNK = PW // C

_LN2 = 0.6931471805599453
_SQRT2 = 1.4142135623730951


def _eval_vec(f):
    x = f * jnp.float32(1e-9)

    b0 = (x >= 1.0) & (x < 10.0)
    b1 = (x > 10.0) & (x < 100.0)
    b2 = (x > 100.0) & (x <= 1000.0)

    bits = lax.bitcast_convert_type(x, jnp.int32)
    e = (bits >> 23) - 127
    m = lax.bitcast_convert_type(
        (bits & 0x007FFFFF) | 0x3F800000, jnp.float32)
    big = m > _SQRT2
    m = jnp.where(big, m * 0.5, m)
    e = jnp.where(big, e + 1, e)
    s = (m - 1.0) / (m + 1.0)
    s2 = s * s
    p = 1.0 + s2 * (0.3333333333333333 + s2 * (0.2 + s2 * 0.14285714285714285))
    lnx = e.astype(jnp.float32) * jnp.float32(_LN2) + 2.0 * s * p

    a = jnp.where(b0, 3.0, jnp.where(b1, 15.0, jnp.where(b2, 30.0, -1.0)))
    b = jnp.where(b1, -0.1, jnp.where(b2, -0.4, 0.0))
    c = jnp.where(b0, 1.5e-4, jnp.where(b1, 0.035, jnp.where(b2, 0.15, -1.0)))
    d = jnp.where(b0, 2.52, jnp.where(b1, 1.63, jnp.where(b2, 1.30, 0.0)))

    rel = a * jnp.exp(b * lnx)
    cond = c * jnp.exp(d * lnx)
    return rel, cond


@functools.cache
def _build_sc_material():
    @functools.partial(
        pl.kernel,
        out_type=(
            jax.ShapeDtypeStruct((N,), jnp.float32),
            jax.ShapeDtypeStruct((N,), jnp.float32),
        ),
        mesh=plsc.VectorSubcoreMesh(
            core_axis_name="c", subcore_axis_name="s",
            num_cores=NC, num_subcores=NS),
        scratch_types=[
            pltpu.VMEM((C,), jnp.float32),
            pltpu.VMEM((C,), jnp.float32),
            pltpu.VMEM((C,), jnp.float32),
        ],
    )
    def _sc_material(freq_hbm, rel_hbm, cond_hbm, in_v, rel_v, cond_v):
        wid = lax.axis_index("s") * NC + lax.axis_index("c")
        base0 = wid * PW

        def chunk_body(k, carry):
            base = base0 + k * C
            pltpu.sync_copy(freq_hbm.at[pl.ds(base, C)], in_v)

            @plsc.parallel_loop(0, C, L)
            def _(i):
                rel, cond = _eval_vec(in_v[pl.ds(i, L)])
                rel_v[pl.ds(i, L)] = rel
                cond_v[pl.ds(i, L)] = cond

            pltpu.sync_copy(rel_v, rel_hbm.at[pl.ds(base, C)])
            pltpu.sync_copy(cond_v, cond_hbm.at[pl.ds(base, C)])
            return carry

        lax.fori_loop(0, NCHUNK, chunk_body, 0)

    return _sc_material


def kernel(frequency):
    return _build_sc_material()(frequency)


# speedup vs baseline: 2.0117x; 1.4328x over previous
import functools

import jax
import jax.numpy as jnp
from jax import lax
from jax.experimental import pallas as pl
from jax.experimental.pallas import tpu as pltpu
from jax.experimental.pallas import tpu_sc as plsc

N = 16777216
NC, NS, L = 2, 16, 16
NW = NC * NS
PW = N // NW
C = 16384
NCHUNK = PW // C

_LN2 = 0.6931471805599453

_A = (3.0, 15.0, 30.0, -1.0)
_B = (0.0, -0.1, -0.4, 0.0)
_C = (1.5e-4, 0.035, 0.15, -1.0)
_D = (2.52, 1.63, 1.30, 0.0)

_BAND_BY_SLOT = (0, 3, 1, 1, 3, 2, 2, 2, 3, 0, 1, 2, 3, 3, 3, 3)
_NEG = float("-inf")
_THR_TAB = (_NEG, _NEG, _NEG, _NEG, 10.0, _NEG, _NEG, 100.0,
            _NEG, _NEG, 1000.0, _NEG, _NEG, _NEG, _NEG, _NEG)
_BASE_TAB = (10, 7, 7, 7, 0, 8, 8, 3, 9, 9, 6, 10, 0, 0, 0, 0)

_P6 = (-8.900908136655339e-07, 1.4426748671815761, -0.7211167176246251,
       0.4819795393179503, -0.37047476258227585, 0.2876216675908218,
       -0.13948653592849916)


def _table_inputs():
    rows = [
        _THR_TAB,
        [_A[i] for i in _BAND_BY_SLOT],
        [_B[i] * _LN2 for i in _BAND_BY_SLOT],
        [_C[i] for i in _BAND_BY_SLOT],
        [_D[i] * _LN2 for i in _BAND_BY_SLOT],
    ]
    return (jnp.asarray(rows, dtype=jnp.float32),
            jnp.asarray(_BASE_TAB, dtype=jnp.int32))


def _take(vec, idx):
    return jnp.take_along_axis(vec, idx, axis=0, mode="promise_in_bounds")


def _eval_vec(f, thr_v, base_v, a_v, bln_v, c_v, dln_v):
    x = f * jnp.float32(1e-9)

    bits = lax.bitcast_convert_type(x, jnp.int32)

    idx_bin = (bits >> 23) - 126
    thr = _take(thr_v, idx_bin)
    one = jnp.ones_like(idx_bin)
    g = jnp.where(x > thr, 2 * one, jnp.where(x >= thr, one, 0 * one))
    slot = _take(base_v, idx_bin) + g
    a = _take(a_v, slot)
    bln = _take(bln_v, slot)
    c = _take(c_v, slot)
    dln = _take(dln_v, slot)

    ebr = (bits + 0x00400000) >> 23
    scale = lax.bitcast_convert_type((254 - ebr) << 23, jnp.float32)
    t = x * scale - 1.0
    p = jnp.float32(_P6[6])
    p = p * t + jnp.float32(_P6[5])
    p = p * t + jnp.float32(_P6[4])
    p = p * t + jnp.float32(_P6[3])
    p = p * t + jnp.float32(_P6[2])
    p = p * t + jnp.float32(_P6[1])
    p = p * t + jnp.float32(_P6[0])
    lg2 = p + (ebr - 127).astype(jnp.float32)

    rel = a * jnp.exp(bln * lg2)
    cond = c * jnp.exp(dln * lg2)
    return rel, cond


@functools.cache
def _build_sc_material():
    @functools.partial(
        pl.kernel,
        out_type=(
            jax.ShapeDtypeStruct((N,), jnp.float32),
            jax.ShapeDtypeStruct((N,), jnp.float32),
        ),
        mesh=plsc.VectorSubcoreMesh(
            core_axis_name="c", subcore_axis_name="s",
            num_cores=NC, num_subcores=NS),
        scratch_types=[
            pltpu.VMEM((5, L), jnp.float32), pltpu.VMEM((L,), jnp.int32),
            pltpu.VMEM((C,), jnp.float32), pltpu.VMEM((C,), jnp.float32),
            pltpu.VMEM((C,), jnp.float32), pltpu.VMEM((C,), jnp.float32),
            pltpu.VMEM((C,), jnp.float32), pltpu.VMEM((C,), jnp.float32),
            pltpu.SemaphoreType.DMA, pltpu.SemaphoreType.DMA,
            pltpu.SemaphoreType.DMA, pltpu.SemaphoreType.DMA,
        ],
    )
    def _sc_material(freq_hbm, ftab_hbm, btab_hbm, rel_hbm, cond_hbm,
                     ftab_v, btab_v,
                     in0, in1, rel0, rel1, cond0, cond1,
                     isem0, isem1, osem0, osem1):
        wid = lax.axis_index("s") * NC + lax.axis_index("c")
        base0 = wid * PW

        def in_slice(k):
            return freq_hbm.at[pl.ds(base0 + k * C, C)]

        pltpu.async_copy(in_slice(0), in0, isem0)
        pltpu.async_copy(in_slice(1), in1, isem1)
        pltpu.sync_copy(ftab_hbm, ftab_v)
        pltpu.sync_copy(btab_hbm, btab_v)

        thr_v = ftab_v[0]
        a_v = ftab_v[1]
        bln_v = ftab_v[2]
        c_v = ftab_v[3]
        dln_v = ftab_v[4]
        base_v = btab_v[...]

        def compute(in_v, rel_v, cond_v):
            @plsc.parallel_loop(0, C, L)
            def _(i):
                rel, cond = _eval_vec(
                    in_v[pl.ds(i, L)],
                    thr_v, base_v, a_v, bln_v, c_v, dln_v)
                rel_v[pl.ds(i, L)] = rel
                cond_v[pl.ds(i, L)] = cond

        def half(kk, k, in_v, rel_v, cond_v, isem, osem):
            base = base0 + k * C
            pltpu.make_async_copy(in_slice(k), in_v, isem).wait()

            @pl.when(kk > 0)
            def _():
                pltpu.make_async_copy(
                    rel_v, rel_hbm.at[pl.ds(base, C)], osem).wait()
                pltpu.make_async_copy(
                    cond_v, cond_hbm.at[pl.ds(base, C)], osem).wait()

            compute(in_v, rel_v, cond_v)
            pltpu.async_copy(rel_v, rel_hbm.at[pl.ds(base, C)], osem)
            pltpu.async_copy(cond_v, cond_hbm.at[pl.ds(base, C)], osem)

            @pl.when(kk < NCHUNK // 2 - 1)
            def _():
                pltpu.async_copy(in_slice(k + 2), in_v, isem)

        def pair_body(kk, carry):
            half(kk, 2 * kk, in0, rel0, cond0, isem0, osem0)
            half(kk, 2 * kk + 1, in1, rel1, cond1, isem1, osem1)
            return carry

        lax.fori_loop(0, NCHUNK // 2, pair_body, 0)

        last0 = base0 + (NCHUNK - 2) * C
        last1 = base0 + (NCHUNK - 1) * C
        pltpu.make_async_copy(rel0, rel_hbm.at[pl.ds(last0, C)], osem0).wait()
        pltpu.make_async_copy(cond0, cond_hbm.at[pl.ds(last0, C)], osem0).wait()
        pltpu.make_async_copy(rel1, rel_hbm.at[pl.ds(last1, C)], osem1).wait()
        pltpu.make_async_copy(cond1, cond_hbm.at[pl.ds(last1, C)], osem1).wait()

    return _sc_material


def kernel(frequency):
    ftab, btab = _table_inputs()
    return _build_sc_material()(frequency, ftab, btab)


# speedup vs baseline: 2.2772x; 1.1319x over previous
import functools

import jax
import jax.numpy as jnp
from jax import lax
from jax.experimental import pallas as pl
from jax.experimental.pallas import tpu as pltpu
from jax.experimental.pallas import tpu_sc as plsc

N = 16777216
NC, NS, L = 2, 16, 16
NW = NC * NS
PW = N // NW
C = 16384
NCHUNK = PW // C

_LN2 = 0.6931471805599453

_A = (3.0, 15.0, 30.0, -1.0)
_B = (0.0, -0.1, -0.4, 0.0)
_C = (1.5e-4, 0.035, 0.15, -1.0)
_D = (2.52, 1.63, 1.30, 0.0)

_BAND_BY_SLOT = (3, 0, 0, 0, 3, 1, 1, 3, 2, 2, 2, 3, 0, 1, 2, 3)
_NEG = float("-inf")
_THR_TAB = (_NEG, 1.0e9, _NEG, _NEG, _NEG, 1.0e10, _NEG, _NEG,
            100000006144.0, _NEG, _NEG, 1.0e12, _NEG, _NEG, _NEG, _NEG)
_BASE_TAB = (13, 0, 10, 10, 10, 3, 11, 11, 6, 12, 12, 9, 13, 0, 0, 0)

_P5 = (-156.89734854740013, 1.4425448373784, -0.7218597211895192,
       0.4899050444148601, -0.3645310015370965, 0.18300676564410928)


def _table_inputs():
    rows = [
        _THR_TAB,
        [_A[i] for i in _BAND_BY_SLOT],
        [_B[i] * _LN2 for i in _BAND_BY_SLOT],
        [_C[i] for i in _BAND_BY_SLOT],
        [_D[i] * _LN2 for i in _BAND_BY_SLOT],
    ]
    return (jnp.asarray(rows, dtype=jnp.float32),
            jnp.asarray(_BASE_TAB, dtype=jnp.int32))


def _take(vec, idx):
    return jnp.take_along_axis(vec, idx, axis=0, mode="promise_in_bounds")


def _eval_vec(f, thr_v, base_v, a_v, bln_v, c_v, dln_v):
    bits = lax.bitcast_convert_type(f, jnp.int32)

    idx_bin = (bits >> 23) - 155
    thr = _take(thr_v, idx_bin)
    one = jnp.ones_like(idx_bin)
    g = jnp.where(f > thr, 2 * one, jnp.where(f >= thr, one, 0 * one))
    slot = _take(base_v, idx_bin) + g
    a = _take(a_v, slot)
    bln = _take(bln_v, slot)
    c = _take(c_v, slot)
    dln = _take(dln_v, slot)

    ebr = (bits + 0x00400000) >> 23
    scale = lax.bitcast_convert_type((254 - ebr) << 23, jnp.float32)
    t = f * scale - 1.0
    p = jnp.float32(_P5[5])
    p = p * t + jnp.float32(_P5[4])
    p = p * t + jnp.float32(_P5[3])
    p = p * t + jnp.float32(_P5[2])
    p = p * t + jnp.float32(_P5[1])
    p = p * t + jnp.float32(_P5[0])
    lg2 = p + ebr.astype(jnp.float32)

    rel = a * jnp.exp(bln * lg2)
    cond = c * jnp.exp(dln * lg2)
    return rel, cond


@functools.cache
def _build_sc_material():
    @functools.partial(
        pl.kernel,
        out_type=(
            jax.ShapeDtypeStruct((N,), jnp.float32),
            jax.ShapeDtypeStruct((N,), jnp.float32),
        ),
        mesh=plsc.VectorSubcoreMesh(
            core_axis_name="c", subcore_axis_name="s",
            num_cores=NC, num_subcores=NS),
        scratch_types=[
            pltpu.VMEM((5, L), jnp.float32), pltpu.VMEM((L,), jnp.int32),
            pltpu.VMEM((C,), jnp.float32), pltpu.VMEM((C,), jnp.float32),
            pltpu.VMEM((C,), jnp.float32), pltpu.VMEM((C,), jnp.float32),
            pltpu.VMEM((C,), jnp.float32), pltpu.VMEM((C,), jnp.float32),
            pltpu.SemaphoreType.DMA, pltpu.SemaphoreType.DMA,
            pltpu.SemaphoreType.DMA, pltpu.SemaphoreType.DMA,
        ],
    )
    def _sc_material(freq_hbm, ftab_hbm, btab_hbm, rel_hbm, cond_hbm,
                     ftab_v, btab_v,
                     in0, in1, rel0, rel1, cond0, cond1,
                     isem0, isem1, osem0, osem1):
        wid = lax.axis_index("s") * NC + lax.axis_index("c")
        base0 = wid * PW

        def in_slice(k):
            return freq_hbm.at[pl.ds(base0 + k * C, C)]

        pltpu.async_copy(in_slice(0), in0, isem0)
        pltpu.async_copy(in_slice(1), in1, isem1)
        pltpu.sync_copy(ftab_hbm, ftab_v)
        pltpu.sync_copy(btab_hbm, btab_v)

        thr_v = ftab_v[0]
        a_v = ftab_v[1]
        bln_v = ftab_v[2]
        c_v = ftab_v[3]
        dln_v = ftab_v[4]
        base_v = btab_v[...]

        def compute(in_v, rel_v, cond_v):
            @plsc.parallel_loop(0, C, L)
            def _(i):
                rel, cond = _eval_vec(
                    in_v[pl.ds(i, L)],
                    thr_v, base_v, a_v, bln_v, c_v, dln_v)
                rel_v[pl.ds(i, L)] = rel
                cond_v[pl.ds(i, L)] = cond

        def half(kk, k, in_v, rel_v, cond_v, isem, osem):
            base = base0 + k * C
            pltpu.make_async_copy(in_slice(k), in_v, isem).wait()

            @pl.when(kk > 0)
            def _():
                pltpu.make_async_copy(
                    rel_v, rel_hbm.at[pl.ds(base, C)], osem).wait()
                pltpu.make_async_copy(
                    cond_v, cond_hbm.at[pl.ds(base, C)], osem).wait()

            compute(in_v, rel_v, cond_v)
            pltpu.async_copy(rel_v, rel_hbm.at[pl.ds(base, C)], osem)
            pltpu.async_copy(cond_v, cond_hbm.at[pl.ds(base, C)], osem)

            @pl.when(kk < NCHUNK // 2 - 1)
            def _():
                pltpu.async_copy(in_slice(k + 2), in_v, isem)

        def pair_body(kk, carry):
            half(kk, 2 * kk, in0, rel0, cond0, isem0, osem0)
            half(kk, 2 * kk + 1, in1, rel1, cond1, isem1, osem1)
            return carry

        lax.fori_loop(0, NCHUNK // 2, pair_body, 0)

        last0 = base0 + (NCHUNK - 2) * C
        last1 = base0 + (NCHUNK - 1) * C
        pltpu.make_async_copy(rel0, rel_hbm.at[pl.ds(last0, C)], osem0).wait()
        pltpu.make_async_copy(cond0, cond_hbm.at[pl.ds(last0, C)], osem0).wait()
        pltpu.make_async_copy(rel1, rel_hbm.at[pl.ds(last1, C)], osem1).wait()
        pltpu.make_async_copy(cond1, cond_hbm.at[pl.ds(last1, C)], osem1).wait()

    return _sc_material


def kernel(frequency):
    ftab, btab = _table_inputs()
    return _build_sc_material()(frequency, ftab, btab)


# speedup vs baseline: 2.4356x; 1.0696x over previous
import functools

import jax
import jax.numpy as jnp
from jax import lax
from jax.experimental import pallas as pl
from jax.experimental.pallas import tpu as pltpu
from jax.experimental.pallas import tpu_sc as plsc

N = 16777216
NC, NS, L = 2, 16, 16
NW = NC * NS
PW = N // NW
C = 16384
NCHUNK = PW // C

_LN2 = 0.6931471805599453

_A = (3.0, 15.0, 30.0, -1.0)
_B = (0.0, -0.1, -0.4, 0.0)
_C = (1.5e-4, 0.035, 0.15, -1.0)
_D = (2.52, 1.63, 1.30, 0.0)

_BAND_BY_SLOT = (3, 0, 0, 0, 3, 1, 1, 3, 2, 2, 2, 3, 0, 1, 2, 3)
_NEG = float("-inf")
_THR_TAB = (_NEG, 1.0e9, _NEG, _NEG, _NEG, 1.0e10, _NEG, _NEG,
            100000006144.0, _NEG, _NEG, 1.0e12, _NEG, _NEG, _NEG, _NEG)
_BASE_TAB = (13, 0, 10, 10, 10, 3, 11, 11, 6, 12, 12, 9, 13, 0, 0, 0)

_P4 = (-159.80915647634861, 5.371138987534238, -3.6996336354567507,
       1.4905116583281666, -0.2501516357927904)


def _table_inputs():
    rows = [
        _THR_TAB,
        [_A[i] for i in _BAND_BY_SLOT],
        [_B[i] * _LN2 for i in _BAND_BY_SLOT],
        [_C[i] for i in _BAND_BY_SLOT],
        [_D[i] * _LN2 for i in _BAND_BY_SLOT],
    ]
    return (jnp.asarray(rows, dtype=jnp.float32),
            jnp.asarray(_BASE_TAB, dtype=jnp.int32))


def _take(vec, idx):
    return jnp.take_along_axis(vec, idx, axis=0, mode="promise_in_bounds")


def _eval_vec(f, thr_v, base_v, a_v, bln_v, c_v, dln_v):
    bits = lax.bitcast_convert_type(f, jnp.int32)

    idx_bin = (bits >> 23) - 155
    thr = _take(thr_v, idx_bin)
    one = jnp.ones_like(idx_bin)
    g = jnp.where(f > thr, 2 * one, jnp.where(f >= thr, one, 0 * one))
    slot = _take(base_v, idx_bin) + g
    a = _take(a_v, slot)
    bln = _take(bln_v, slot)
    c = _take(c_v, slot)
    dln = _take(dln_v, slot)

    ebr = (bits + 0x00400000) >> 23
    scale = lax.bitcast_convert_type((254 - ebr) << 23, jnp.float32)
    m = f * scale
    p = jnp.float32(_P4[4])
    p = p * m + jnp.float32(_P4[3])
    p = p * m + jnp.float32(_P4[2])
    p = p * m + jnp.float32(_P4[1])
    p = p * m + jnp.float32(_P4[0])
    lg2 = p + ebr.astype(jnp.float32)

    rel = a * jnp.exp(bln * lg2)
    cond = c * jnp.exp(dln * lg2)
    return rel, cond


@functools.cache
def _build_sc_material():
    @functools.partial(
        pl.kernel,
        out_type=(
            jax.ShapeDtypeStruct((N,), jnp.float32),
            jax.ShapeDtypeStruct((N,), jnp.float32),
        ),
        mesh=plsc.VectorSubcoreMesh(
            core_axis_name="c", subcore_axis_name="s",
            num_cores=NC, num_subcores=NS),
        scratch_types=[
            pltpu.VMEM((5, L), jnp.float32), pltpu.VMEM((L,), jnp.int32),
            pltpu.VMEM((C,), jnp.float32), pltpu.VMEM((C,), jnp.float32),
            pltpu.VMEM((C,), jnp.float32), pltpu.VMEM((C,), jnp.float32),
            pltpu.VMEM((C,), jnp.float32), pltpu.VMEM((C,), jnp.float32),
            pltpu.SemaphoreType.DMA, pltpu.SemaphoreType.DMA,
            pltpu.SemaphoreType.DMA, pltpu.SemaphoreType.DMA,
        ],
    )
    def _sc_material(freq_hbm, ftab_hbm, btab_hbm, rel_hbm, cond_hbm,
                     ftab_v, btab_v,
                     in0, in1, rel0, rel1, cond0, cond1,
                     isem0, isem1, osem0, osem1):
        wid = lax.axis_index("s") * NC + lax.axis_index("c")
        base0 = wid * PW

        def in_slice(k):
            return freq_hbm.at[pl.ds(base0 + k * C, C)]

        pltpu.async_copy(in_slice(0), in0, isem0)
        pltpu.async_copy(in_slice(1), in1, isem1)
        pltpu.sync_copy(ftab_hbm, ftab_v)
        pltpu.sync_copy(btab_hbm, btab_v)

        thr_v = ftab_v[0]
        a_v = ftab_v[1]
        bln_v = ftab_v[2]
        c_v = ftab_v[3]
        dln_v = ftab_v[4]
        base_v = btab_v[...]

        def compute(in_v, rel_v, cond_v):
            @plsc.parallel_loop(0, C, L)
            def _(i):
                rel, cond = _eval_vec(
                    in_v[pl.ds(i, L)],
                    thr_v, base_v, a_v, bln_v, c_v, dln_v)
                rel_v[pl.ds(i, L)] = rel
                cond_v[pl.ds(i, L)] = cond

        def half(kk, k, in_v, rel_v, cond_v, isem, osem):
            base = base0 + k * C
            pltpu.make_async_copy(in_slice(k), in_v, isem).wait()

            @pl.when(kk > 0)
            def _():
                pltpu.make_async_copy(
                    rel_v, rel_hbm.at[pl.ds(base, C)], osem).wait()
                pltpu.make_async_copy(
                    cond_v, cond_hbm.at[pl.ds(base, C)], osem).wait()

            compute(in_v, rel_v, cond_v)
            pltpu.async_copy(rel_v, rel_hbm.at[pl.ds(base, C)], osem)
            pltpu.async_copy(cond_v, cond_hbm.at[pl.ds(base, C)], osem)

            @pl.when(kk < NCHUNK // 2 - 1)
            def _():
                pltpu.async_copy(in_slice(k + 2), in_v, isem)

        def pair_body(kk, carry):
            half(kk, 2 * kk, in0, rel0, cond0, isem0, osem0)
            half(kk, 2 * kk + 1, in1, rel1, cond1, isem1, osem1)
            return carry

        lax.fori_loop(0, NCHUNK // 2, pair_body, 0)

        last0 = base0 + (NCHUNK - 2) * C
        last1 = base0 + (NCHUNK - 1) * C
        pltpu.make_async_copy(rel0, rel_hbm.at[pl.ds(last0, C)], osem0).wait()
        pltpu.make_async_copy(cond0, cond_hbm.at[pl.ds(last0, C)], osem0).wait()
        pltpu.make_async_copy(rel1, rel_hbm.at[pl.ds(last1, C)], osem1).wait()
        pltpu.make_async_copy(cond1, cond_hbm.at[pl.ds(last1, C)], osem1).wait()

    return _sc_material


def kernel(frequency):
    ftab, btab = _table_inputs()
    return _build_sc_material()(frequency, ftab, btab)
